# baseline (speedup 1.0000x reference)
import dataclasses
import functools

import jax
import jax.numpy as jnp
from jax import lax
from jax.experimental import pallas as pl
from jax.experimental.pallas import tpu as pltpu
from jax.experimental.pallas import tpu_sc as plsc

B, L, V, D = 1024, 50, 100000, 1024
D_TOPIC, D_F = 128, 128
NC, NS, LANES = 2, 16, 16
NW = NC * NS
ROWS_PW = B // NW
IDS_PW = ROWS_PW * L
WPAD = 64
GCHUNK = 24
NCHUNK = 2
NSEG = 4
SEG = 25600
VPAD = NSEG * SEG

_sc_mesh = plsc.VectorSubcoreMesh(core_axis_name="c", subcore_axis_name="s")

_sc_params = pltpu.CompilerParams()
if "needs_layout_passes" in pltpu.CompilerParams.__dataclass_fields__:
    _sc_params = dataclasses.replace(_sc_params, needs_layout_passes=False)


@functools.partial(
    pl.kernel,
    out_type=(
        jax.ShapeDtypeStruct((B, D), jnp.float32),
        jax.ShapeDtypeStruct((B, LANES), jnp.float32),
    ),
    mesh=_sc_mesh,
    compiler_params=_sc_params,
    scratch_types=[
        pltpu.VMEM((ROWS_PW, WPAD), jnp.int32),
        pltpu.VMEM((ROWS_PW * WPAD + LANES,), jnp.float32),
        pltpu.VMEM((SEG,), jnp.float32),
        pltpu.VMEM((GCHUNK, D), jnp.float32),
        pltpu.VMEM((GCHUNK, D), jnp.float32),
        pltpu.VMEM((2, D), jnp.float32),
        pltpu.VMEM((ROWS_PW, D), jnp.float32),
        pltpu.VMEM((ROWS_PW, LANES), jnp.float32),
        pltpu.SemaphoreType.DMA,
        pltpu.SemaphoreType.DMA,
        pltpu.SemaphoreType.DMA,
    ],
)
def _centroid_sc(ids_hbm, idf_hbm, wte_hbm, raw_hbm, wpart_hbm,
                 ids_v, w_v, seg_v, bufA_v, bufB_v, buf2_v, rows_v,
                 wpart_v, semA, semB, sem2):
    wid = lax.axis_index("s") * NC + lax.axis_index("c")
    row0 = wid * ROWS_PW

    pltpu.sync_copy(ids_hbm.at[pl.ds(row0, ROWS_PW)], ids_v)

    gbufs = (bufA_v, bufB_v)
    gsems = (semA, semB)

    def _chunk_copy(r, i):
        return (wte_hbm.at[ids_v.at[r, pl.ds(i * GCHUNK, GCHUNK)]],
                gbufs[i], gsems[i])

    def _tail_copy(r):
        return (wte_hbm.at[ids_v.at[r, pl.ds(NCHUNK * GCHUNK, 2)]],
                buf2_v, sem2)

    def gather_chunk(r, i):
        pltpu.async_copy(*_chunk_copy(r, i))

    def wait_chunk(r, i):
        pltpu.make_async_copy(*_chunk_copy(r, i)).wait()

    def gather_tail(r):
        pltpu.async_copy(*_tail_copy(r))

    def wait_tail(r):
        pltpu.make_async_copy(*_tail_copy(r)).wait()

    gather_chunk(0, 0)
    gather_chunk(0, 1)
    gather_tail(0)

    for s in range(NSEG):
        pltpu.sync_copy(idf_hbm.at[pl.ds(s * SEG, SEG)], seg_v)

        @pl.loop(0, ROWS_PW)
        def _(r):
            for k in range(4):
                off = r * WPAD + k * LANES
                rel = ids_v[r, pl.ds(k * LANES, LANES)] - (s * SEG)
                valid = (rel >= 0) & (rel < SEG)
                relc = jnp.clip(rel, 0, SEG - 1)
                vals = plsc.load_gather(seg_v, [relc])
                prev = (jnp.zeros((LANES,), jnp.float32) if s == 0
                        else w_v[pl.ds(off, LANES)])
                w_v[pl.ds(off, LANES)] = jnp.where(valid, vals, prev)

    lane_lt2 = lax.iota(jnp.int32, LANES) < 2

    @pl.loop(0, ROWS_PW)
    def _(r):
        ssum = jnp.zeros((LANES,), jnp.float32)
        for k in range(4):
            off = r * WPAD + k * LANES
            w = jnp.maximum(w_v[pl.ds(off, LANES)], 0.1)
            if k == 3:
                w = jnp.where(lane_lt2, w, 0.0)
            w_v[pl.ds(off, LANES)] = w
            ssum = ssum + w
        wpart_v[r, :] = ssum

    def accum(r, tbase, nl, src, first):
        wls = [lax.broadcast_in_dim(
                   w_v[pl.ds(r * WPAD + tbase + l, LANES)][0], (LANES,), ())
               for l in range(nl)]

        @plsc.parallel_loop(0, D, step=LANES, unroll=2)
        def _(c):
            terms = [src[l, pl.ds(c, LANES)] * wls[l] for l in range(nl)]
            while len(terms) > 1:
                terms = [terms[i] + terms[i + 1] if i + 1 < len(terms)
                         else terms[i] for i in range(0, len(terms), 2)]
            if first:
                rows_v[r, pl.ds(c, LANES)] = terms[0]
            else:
                plsc.addupdate(rows_v.at[r, pl.ds(c, LANES)], terms[0])

    def do_row(r, last):
        for i in range(NCHUNK):
            wait_chunk(r, i)
            accum(r, i * GCHUNK, GCHUNK, gbufs[i], first=(i == 0))
            if not last:
                gather_chunk(r + 1, i)
        wait_tail(r)
        accum(r, NCHUNK * GCHUNK, 2, buf2_v, first=False)
        if not last:
            gather_tail(r + 1)

    @pl.loop(0, ROWS_PW - 1)
    def _(r):
        do_row(r, last=False)

    do_row(ROWS_PW - 1, last=True)

    pltpu.sync_copy(rows_v, raw_hbm.at[pl.ds(row0, ROWS_PW)])
    pltpu.sync_copy(wpart_v, wpart_hbm.at[pl.ds(row0, ROWS_PW)])


def _silu(x):
    return x / (1.0 + jnp.exp(-x))


def _pre_body(h_ref, Wh_ref, bh_ref, Wf1h_ref, bf1_ref, hWh_ref, hWf1_ref):
    h = h_ref[...]
    hWh_ref[...] = (jnp.dot(h, Wh_ref[...], preferred_element_type=jnp.float32)
                    + bh_ref[...])
    hWf1_ref[...] = (jnp.dot(h, Wf1h_ref[...], preferred_element_type=jnp.float32)
                     + bf1_ref[...])


_pre = pl.pallas_call(
    _pre_body,
    out_shape=(
        jax.ShapeDtypeStruct((B, D_TOPIC), jnp.float32),
        jax.ShapeDtypeStruct((B, 4 * D_F), jnp.float32),
    ),
)


def _mlp_body(raw_ref, wpart_ref, hWh_ref, hWf1_ref, W1_ref, b1_ref, W2_ref,
              b2_ref, Wf1b_ref, Wf2_ref, bf2_ref, base_ref, fiber_ref):
    wsum = jnp.sum(wpart_ref[...], axis=1, keepdims=True)
    cent = raw_ref[...] / jnp.maximum(wsum, 1e-8)
    t1 = _silu(jnp.dot(cent, W1_ref[...], preferred_element_type=jnp.float32)
               + b1_ref[...])
    down = jnp.dot(t1, W2_ref[...], preferred_element_type=jnp.float32) + b2_ref[...]
    mixed = down + hWh_ref[...]
    norm = jnp.sqrt(jnp.sum(mixed * mixed, axis=1, keepdims=True))
    base = mixed / jnp.maximum(norm, 1e-8)
    base_ref[...] = base
    f1 = _silu(jnp.dot(base, Wf1b_ref[...], preferred_element_type=jnp.float32)
               + hWf1_ref[...])
    fiber_ref[...] = (jnp.dot(f1, Wf2_ref[...], preferred_element_type=jnp.float32)
                      + bf2_ref[...])


_mlp = pl.pallas_call(
    _mlp_body,
    out_shape=(
        jax.ShapeDtypeStruct((B, D_TOPIC), jnp.float32),
        jax.ShapeDtypeStruct((B, D_F), jnp.float32),
    ),
)


def kernel(hidden_state, content_token_ids, wte_normed, idf,
           W1, b1, W2, b2, Wh, bh, Wf1, bf1, Wf2, bf2):
    ids = jnp.pad(content_token_ids.astype(jnp.int32), ((0, 0), (0, WPAD - L)))
    idf_pad = jnp.pad(idf, (0, VPAD - V))
    raw, wpart = _centroid_sc(ids, idf_pad, wte_normed)
    hWh, hWf1 = _pre(hidden_state, Wh, bh.reshape(1, -1),
                     Wf1[:D], bf1.reshape(1, -1))
    base, fiber = _mlp(raw, wpart, hWh, hWf1,
                       W1, b1.reshape(1, -1), W2, b2.reshape(1, -1),
                       Wf1[D:], Wf2, bf2.reshape(1, -1))
    return (base, fiber, base)


# speedup vs baseline: 1.1242x; 1.1242x over previous
import dataclasses
import functools

import jax
import jax.numpy as jnp
from jax import lax
from jax.experimental import pallas as pl
from jax.experimental.pallas import tpu as pltpu
from jax.experimental.pallas import tpu_sc as plsc

B, L, V, D = 1024, 50, 100000, 1024
D_TOPIC, D_F = 128, 128
NC, NS, LANES = 2, 16, 16
NW = NC * NS
ROWS_PW = B // NW
IDS_PW = ROWS_PW * L
WPAD = 64
GCHUNK = 8
NCHUNK = 6
NSEG = 8
SEG = 12800
VPAD = NSEG * SEG

_sc_mesh = plsc.VectorSubcoreMesh(core_axis_name="c", subcore_axis_name="s")

_sc_params = pltpu.CompilerParams()
if "needs_layout_passes" in pltpu.CompilerParams.__dataclass_fields__:
    _sc_params = dataclasses.replace(_sc_params, needs_layout_passes=False)


@functools.partial(
    pl.kernel,
    out_type=(
        jax.ShapeDtypeStruct((B, D), jnp.float32),
        jax.ShapeDtypeStruct((B, LANES), jnp.float32),
    ),
    mesh=_sc_mesh,
    compiler_params=_sc_params,
    scratch_types=[
        pltpu.VMEM((ROWS_PW, WPAD), jnp.int32),
        pltpu.VMEM((ROWS_PW * WPAD + LANES,), jnp.float32),
        pltpu.VMEM((SEG,), jnp.float32),
        pltpu.VMEM((SEG,), jnp.float32),
        pltpu.VMEM((GCHUNK, D), jnp.float32),
        pltpu.VMEM((GCHUNK, D), jnp.float32),
        pltpu.VMEM((GCHUNK, D), jnp.float32),
        pltpu.VMEM((GCHUNK, D), jnp.float32),
        pltpu.VMEM((GCHUNK, D), jnp.float32),
        pltpu.VMEM((GCHUNK, D), jnp.float32),
        pltpu.VMEM((2, D), jnp.float32),
        pltpu.VMEM((ROWS_PW, D), jnp.float32),
        pltpu.VMEM((ROWS_PW, LANES), jnp.float32),
        pltpu.SemaphoreType.DMA,
        pltpu.SemaphoreType.DMA,
        pltpu.SemaphoreType.DMA,
        pltpu.SemaphoreType.DMA,
        pltpu.SemaphoreType.DMA,
        pltpu.SemaphoreType.DMA,
        pltpu.SemaphoreType.DMA,
        pltpu.SemaphoreType.DMA,
        pltpu.SemaphoreType.DMA,
    ],
)
def _centroid_sc(ids_hbm, idf_hbm, wte_hbm, raw_hbm, wpart_hbm,
                 ids_v, w_v, segA_v, segB_v, buf0_v, buf1_v, buf2w_v, buf3_v,
                 buf4_v, buf5_v, buft_v, rows_v, wpart_v, sem0, sem1, sem2c,
                 sem3, sem4, sem5, semt, ssemA, ssemB):
    wid = lax.axis_index("s") * NC + lax.axis_index("c")
    row0 = wid * ROWS_PW

    pltpu.sync_copy(ids_hbm.at[pl.ds(row0, ROWS_PW)], ids_v)

    gbufs = (buf0_v, buf1_v, buf2w_v, buf3_v, buf4_v, buf5_v)
    gsems = (sem0, sem1, sem2c, sem3, sem4, sem5)
    segbufs = (segA_v, segB_v)
    ssems = (ssemA, ssemB)

    def _chunk_copy(r, i):
        return (wte_hbm.at[ids_v.at[r, pl.ds(i * GCHUNK, GCHUNK)]],
                gbufs[i], gsems[i])

    def _tail_copy(r):
        return (wte_hbm.at[ids_v.at[r, pl.ds(NCHUNK * GCHUNK, 2)]],
                buft_v, semt)

    def gather_chunk(r, i):
        pltpu.async_copy(*_chunk_copy(r, i))

    def wait_chunk(r, i):
        pltpu.make_async_copy(*_chunk_copy(r, i)).wait()

    def gather_tail(r):
        pltpu.async_copy(*_tail_copy(r))

    def wait_tail(r):
        pltpu.make_async_copy(*_tail_copy(r)).wait()

    for i in range(NCHUNK):
        gather_chunk(0, i)
    gather_tail(0)

    def _seg_copy(s):
        return (idf_hbm.at[pl.ds(s * SEG, SEG)], segbufs[s % 2], ssems[s % 2])

    pltpu.async_copy(*_seg_copy(0))
    for s in range(NSEG):
        if s + 1 < NSEG:
            pltpu.async_copy(*_seg_copy(s + 1))
        pltpu.make_async_copy(*_seg_copy(s)).wait()
        seg_v = segbufs[s % 2]

        @pl.loop(0, ROWS_PW)
        def _(r):
            for k in range(4):
                off = r * WPAD + k * LANES
                rel = ids_v[r, pl.ds(k * LANES, LANES)] - (s * SEG)
                valid = (rel >= 0) & (rel < SEG)
                relc = jnp.clip(rel, 0, SEG - 1)
                vals = plsc.load_gather(seg_v, [relc])
                prev = (jnp.zeros((LANES,), jnp.float32) if s == 0
                        else w_v[pl.ds(off, LANES)])
                w_v[pl.ds(off, LANES)] = jnp.where(valid, vals, prev)

    lane_lt2 = lax.iota(jnp.int32, LANES) < 2

    @pl.loop(0, ROWS_PW)
    def _(r):
        ssum = jnp.zeros((LANES,), jnp.float32)
        for k in range(4):
            off = r * WPAD + k * LANES
            w = jnp.maximum(w_v[pl.ds(off, LANES)], 0.1)
            if k == 3:
                w = jnp.where(lane_lt2, w, 0.0)
            w_v[pl.ds(off, LANES)] = w
            ssum = ssum + w
        wpart_v[r, :] = ssum

    def accum(r, tbase, nl, src, first):
        wls = [lax.broadcast_in_dim(
                   w_v[pl.ds(r * WPAD + tbase + l, LANES)][0], (LANES,), ())
               for l in range(nl)]

        @plsc.parallel_loop(0, D, step=LANES, unroll=2)
        def _(c):
            terms = [src[l, pl.ds(c, LANES)] * wls[l] for l in range(nl)]
            while len(terms) > 1:
                terms = [terms[i] + terms[i + 1] if i + 1 < len(terms)
                         else terms[i] for i in range(0, len(terms), 2)]
            if first:
                rows_v[r, pl.ds(c, LANES)] = terms[0]
            else:
                plsc.addupdate(rows_v.at[r, pl.ds(c, LANES)], terms[0])

    def do_row(r, last):
        for i in range(NCHUNK):
            wait_chunk(r, i)
            accum(r, i * GCHUNK, GCHUNK, gbufs[i], first=(i == 0))
            if not last:
                gather_chunk(r + 1, i)
        wait_tail(r)
        accum(r, NCHUNK * GCHUNK, 2, buft_v, first=False)
        if not last:
            gather_tail(r + 1)

    @pl.loop(0, ROWS_PW - 1)
    def _(r):
        do_row(r, last=False)

    do_row(ROWS_PW - 1, last=True)

    pltpu.sync_copy(rows_v, raw_hbm.at[pl.ds(row0, ROWS_PW)])
    pltpu.sync_copy(wpart_v, wpart_hbm.at[pl.ds(row0, ROWS_PW)])


def _silu(x):
    return x / (1.0 + jnp.exp(-x))


def _pre_body(h_ref, Wh_ref, bh_ref, Wf1h_ref, bf1_ref, hWh_ref, hWf1_ref):
    h = h_ref[...]
    hWh_ref[...] = (jnp.dot(h, Wh_ref[...], preferred_element_type=jnp.float32)
                    + bh_ref[...])
    hWf1_ref[...] = (jnp.dot(h, Wf1h_ref[...], preferred_element_type=jnp.float32)
                     + bf1_ref[...])


_pre = pl.pallas_call(
    _pre_body,
    out_shape=(
        jax.ShapeDtypeStruct((B, D_TOPIC), jnp.float32),
        jax.ShapeDtypeStruct((B, 4 * D_F), jnp.float32),
    ),
)


def _mlp_body(raw_ref, wpart_ref, hWh_ref, hWf1_ref, W1_ref, b1_ref, W2_ref,
              b2_ref, Wf1b_ref, Wf2_ref, bf2_ref, base_ref, fiber_ref):
    wsum = jnp.sum(wpart_ref[...], axis=1, keepdims=True)
    cent = raw_ref[...] / jnp.maximum(wsum, 1e-8)
    t1 = _silu(jnp.dot(cent, W1_ref[...], preferred_element_type=jnp.float32)
               + b1_ref[...])
    down = jnp.dot(t1, W2_ref[...], preferred_element_type=jnp.float32) + b2_ref[...]
    mixed = down + hWh_ref[...]
    norm = jnp.sqrt(jnp.sum(mixed * mixed, axis=1, keepdims=True))
    base = mixed / jnp.maximum(norm, 1e-8)
    base_ref[...] = base
    f1 = _silu(jnp.dot(base, Wf1b_ref[...], preferred_element_type=jnp.float32)
               + hWf1_ref[...])
    fiber_ref[...] = (jnp.dot(f1, Wf2_ref[...], preferred_element_type=jnp.float32)
                      + bf2_ref[...])


_mlp = pl.pallas_call(
    _mlp_body,
    out_shape=(
        jax.ShapeDtypeStruct((B, D_TOPIC), jnp.float32),
        jax.ShapeDtypeStruct((B, D_F), jnp.float32),
    ),
)


def kernel(hidden_state, content_token_ids, wte_normed, idf,
           W1, b1, W2, b2, Wh, bh, Wf1, bf1, Wf2, bf2):
    ids = jnp.pad(content_token_ids.astype(jnp.int32), ((0, 0), (0, WPAD - L)))
    idf_pad = jnp.pad(idf, (0, VPAD - V))
    raw, wpart = _centroid_sc(ids, idf_pad, wte_normed)
    hWh, hWf1 = _pre(hidden_state, Wh, bh.reshape(1, -1),
                     Wf1[:D], bf1.reshape(1, -1))
    base, fiber = _mlp(raw, wpart, hWh, hWf1,
                       W1, b1.reshape(1, -1), W2, b2.reshape(1, -1),
                       Wf1[D:], Wf2, bf2.reshape(1, -1))
    return (base, fiber, base)
